# baseline (speedup 1.0000x reference)
import functools

import jax
import jax.numpy as jnp
from jax import lax
from jax.experimental import pallas as pl
from jax.experimental.pallas import tpu as pltpu
from jax.experimental.pallas import tpu_sc as plsc

D = 64
SUB = 128
CHUNK = 256
NSUB = CHUNK // SUB


def kernel(x, table, ln_weight, ln_bias):
    B, H = x.shape
    nrow = B * H
    info = plsc.get_sparse_core_info()
    NC, NS = info.num_cores, info.num_subcores
    NW = NC * NS
    per_w = nrow // NW
    nchunks = per_w // CHUNK
    assert per_w % CHUNK == 0 and nrow % NW == 0

    x2 = x.reshape(nrow).astype(jnp.int32)

    mesh = plsc.VectorSubcoreMesh(core_axis_name="c", subcore_axis_name="s")

    @functools.partial(
        pl.kernel,
        mesh=mesh,
        out_type=jax.ShapeDtypeStruct((nrow, D), jnp.float32),
        compiler_params=pltpu.CompilerParams(
            needs_layout_passes=False, use_tc_tiling_on_sc=False
        ),
        scratch_types=[
            pltpu.VMEM((CHUNK,), jnp.int32),
            pltpu.VMEM((CHUNK, D), jnp.float32),
            pltpu.VMEM((D,), jnp.float32),
            pltpu.VMEM((D,), jnp.float32),
            pltpu.SemaphoreType.DMA,
        ],
    )
    def sc_kernel(x_hbm, tbl_hbm, w_hbm, b_hbm, out_hbm, idx_v, rows_v, w_v, b_v, sem):
        wid = lax.axis_index("s") * NC + lax.axis_index("c")
        pltpu.sync_copy(w_hbm, w_v)
        pltpu.sync_copy(b_hbm, b_v)
        row0 = wid * per_w

        iot = lax.iota(jnp.int32, 16)
        wvecs = [w_v[pl.ds(16 * t, 16)] for t in range(4)]
        bvecs = [b_v[pl.ds(16 * t, 16)] for t in range(4)]
        wsc = [wvecs[j // 16][j % 16] for j in range(D)]
        bsc = [bvecs[j // 16][j % 16] for j in range(D)]

        def chunk_body(g, carry):
            rbase = row0 + g * CHUNK
            pltpu.sync_copy(x_hbm.at[pl.ds(rbase, CHUNK)], idx_v)
            cps = [
                pltpu.async_copy(
                    tbl_hbm.at[idx_v.at[pl.ds(j * SUB, SUB)]],
                    rows_v.at[pl.ds(j * SUB, SUB)],
                    sem,
                )
                for j in range(NSUB)
            ]
            for cp in cps:
                cp.wait()

            def blk_body(t, c):
                rows = t * 16 + iot
                s = jnp.zeros((16,), jnp.float32)
                q = jnp.zeros((16,), jnp.float32)
                for j in range(D):
                    cj = plsc.load_gather(rows_v, [rows, jnp.full((16,), j, jnp.int32)])
                    s = s + cj
                    q = q + cj * cj
                mean = s * (1.0 / D)
                var = q * (1.0 / D) - mean * mean
                xv = var + 1e-5
                ii = plsc.bitcast(xv, jnp.int32)
                ii = 0x5F3759DF - (ii >> 1)
                y = plsc.bitcast(ii, jnp.float32)
                y = y * (1.5 - 0.5 * xv * y * y)
                y = y * (1.5 - 0.5 * xv * y * y)
                y = y * (1.5 - 0.5 * xv * y * y)
                u = mean * y
                for j in range(D):
                    jj = jnp.full((16,), j, jnp.int32)
                    cj = plsc.load_gather(rows_v, [rows, jj])
                    outj = (cj * y - u) * wsc[j] + bsc[j]
                    plsc.store_scatter(rows_v, [rows, jj], outj)
                return c

            lax.fori_loop(0, CHUNK // 16, blk_body, 0)
            pltpu.sync_copy(rows_v, out_hbm.at[pl.ds(rbase, CHUNK)])
            return carry

        lax.fori_loop(0, nchunks, chunk_body, 0)

    out = sc_kernel(x2, table, ln_weight, ln_bias)
    return out.reshape(B, H, D)


# speedup vs baseline: 2.9746x; 2.9746x over previous
import functools

import jax
import jax.numpy as jnp
from jax import lax
from jax.experimental import pallas as pl
from jax.experimental.pallas import tpu as pltpu
from jax.experimental.pallas import tpu_sc as plsc

D = 64
SUB = 128
CHUNK = 256
NSUB = CHUNK // SUB
NBUF = 4


def kernel(x, table, ln_weight, ln_bias):
    B, H = x.shape
    nrow = B * H
    info = plsc.get_sparse_core_info()
    NC, NS = info.num_cores, info.num_subcores
    NW = NC * NS
    per_w = nrow // NW
    nchunks = per_w // CHUNK
    assert per_w % CHUNK == 0 and nrow % NW == 0 and nchunks % NBUF == 0

    x1 = x.reshape(nrow).astype(jnp.int32)

    mesh = plsc.VectorSubcoreMesh(core_axis_name="c", subcore_axis_name="s")

    @functools.partial(
        pl.kernel,
        mesh=mesh,
        out_type=jax.ShapeDtypeStruct((nrow, D), jnp.float32),
        compiler_params=pltpu.CompilerParams(
            needs_layout_passes=False, use_tc_tiling_on_sc=False
        ),
        scratch_types=(
            [pltpu.VMEM((per_w,), jnp.int32)]
            + [pltpu.VMEM((CHUNK, D), jnp.float32) for _ in range(NBUF)]
            + [pltpu.VMEM((D,), jnp.float32) for _ in range(2)]
            + [pltpu.SemaphoreType.DMA for _ in range(2 * NBUF)]
        ),
    )
    def sc_kernel(x_hbm, tbl_hbm, w_hbm, b_hbm, out_hbm, idx_v,
                  r0, r1, r2, r3, w_v, b_v,
                  g0, g1, g2, g3, o0, o1, o2, o3):
        rows_bufs = [r0, r1, r2, r3]
        gsem = [g0, g1, g2, g3]
        osem = [o0, o1, o2, o3]
        wid = lax.axis_index("s") * NC + lax.axis_index("c")
        row0 = wid * per_w
        pltpu.sync_copy(x_hbm.at[pl.ds(row0, per_w)], idx_v)
        pltpu.sync_copy(w_hbm, w_v)
        pltpu.sync_copy(b_hbm, b_v)

        iot = lax.iota(jnp.int32, 16)
        perms = [(iot + sh) & 15 for sh in (8, 4, 2, 1)]
        wvecs = [w_v[pl.ds(16 * t, 16)] for t in range(4)]
        bvecs = [b_v[pl.ds(16 * t, 16)] for t in range(4)]

        def gstart(g, b):
            for j in range(NSUB):
                pltpu.async_copy(
                    tbl_hbm.at[idx_v.at[pl.ds(g * CHUNK + j * SUB, SUB)]],
                    rows_bufs[b].at[pl.ds(j * SUB, SUB)],
                    gsem[b],
                )

        def drain(sem, b):
            pltpu.make_async_copy(
                tbl_hbm.at[pl.ds(0, CHUNK)], rows_bufs[b], sem
            ).wait()

        def compute(rows_ref):
            def row_body(r, c):
                v0 = rows_ref[r, pl.ds(0, 16)]
                v1 = rows_ref[r, pl.ds(16, 16)]
                v2 = rows_ref[r, pl.ds(32, 16)]
                v3 = rows_ref[r, pl.ds(48, 16)]
                s = (v0 + v1) + (v2 + v3)
                q = (v0 * v0 + v1 * v1) + (v2 * v2 + v3 * v3)
                for p in perms:
                    s = s + jnp.take_along_axis(s, p, axis=0,
                                                mode="promise_in_bounds")
                    q = q + jnp.take_along_axis(q, p, axis=0,
                                                mode="promise_in_bounds")
                mean = s * (1.0 / D)
                var = q * (1.0 / D) - mean * mean
                xv = var + 1e-5
                ii = plsc.bitcast(xv, jnp.int32)
                ii = 0x5F3759DF - (ii >> 1)
                y = plsc.bitcast(ii, jnp.float32)
                y = y * (1.5 - 0.5 * xv * y * y)
                y = y * (1.5 - 0.5 * xv * y * y)
                y = y * (1.5 - 0.5 * xv * y * y)
                u = mean * y
                for t, vt in enumerate((v0, v1, v2, v3)):
                    rows_ref[r, pl.ds(16 * t, 16)] = \
                        (vt * y - u) * wvecs[t] + bvecs[t]
                return c

            lax.fori_loop(0, CHUNK, row_body, 0, unroll=4)

        gstart(0, 0)

        def outer(i, carry):
            for b in range(NBUF):
                g = i * NBUF + b
                nb = (b + 1) % NBUF

                @pl.when(g >= NBUF - 1)
                def _():
                    drain(osem[nb], nb)

                @pl.when(g + 1 < nchunks)
                def _():
                    gstart(g + 1, nb)

                drain(gsem[b], b)
                compute(rows_bufs[b])
                pltpu.async_copy(
                    rows_bufs[b],
                    out_hbm.at[pl.ds(row0 + g * CHUNK, CHUNK)],
                    osem[b],
                )
            return carry

        lax.fori_loop(0, nchunks // NBUF, outer, 0)
        for g in range(nchunks - NBUF + 1, nchunks):
            drain(osem[g % NBUF], g % NBUF)

    out = sc_kernel(x1, table, ln_weight, ln_bias)
    return out.reshape(B, H, D)


# speedup vs baseline: 2.9800x; 1.0018x over previous
import functools

import jax
import jax.numpy as jnp
from jax import lax
from jax.experimental import pallas as pl
from jax.experimental.pallas import tpu as pltpu
from jax.experimental.pallas import tpu_sc as plsc

D = 64
NBUF = 4


def kernel(x, table, ln_weight, ln_bias):
    B, H = x.shape
    nrow = B * H
    info = plsc.get_sparse_core_info()
    NC, NS = info.num_cores, info.num_subcores
    NW = NC * NS
    per_w = nrow // NW
    nb_w = B // NW
    subs = [(0, 128), (128, H - 128)]
    assert nrow % NW == 0 and B % NW == 0 and nb_w % NBUF == 0

    x1 = x.reshape(nrow).astype(jnp.int32)

    mesh = plsc.VectorSubcoreMesh(core_axis_name="c", subcore_axis_name="s")

    @functools.partial(
        pl.kernel,
        mesh=mesh,
        out_type=jax.ShapeDtypeStruct((B, H, D), jnp.float32),
        compiler_params=pltpu.CompilerParams(
            needs_layout_passes=False, use_tc_tiling_on_sc=False
        ),
        scratch_types=(
            [pltpu.VMEM((per_w,), jnp.int32)]
            + [pltpu.VMEM((H, D), jnp.float32) for _ in range(NBUF)]
            + [pltpu.VMEM((D,), jnp.float32) for _ in range(2)]
            + [pltpu.SemaphoreType.DMA for _ in range(2 * NBUF)]
        ),
    )
    def sc_kernel(x_hbm, tbl_hbm, w_hbm, b_hbm, out_hbm, idx_v,
                  r0, r1, r2, r3, w_v, b_v,
                  g0, g1, g2, g3, o0, o1, o2, o3):
        rows_bufs = [r0, r1, r2, r3]
        gsem = [g0, g1, g2, g3]
        osem = [o0, o1, o2, o3]
        wid = lax.axis_index("s") * NC + lax.axis_index("c")
        row0 = wid * per_w
        b0 = wid * nb_w
        pltpu.sync_copy(x_hbm.at[pl.ds(row0, per_w)], idx_v)
        pltpu.sync_copy(w_hbm, w_v)
        pltpu.sync_copy(b_hbm, b_v)

        iot = lax.iota(jnp.int32, 16)
        perms = [(iot + sh) & 15 for sh in (8, 4, 2, 1)]
        wvecs = [w_v[pl.ds(16 * t, 16)] for t in range(4)]
        bvecs = [b_v[pl.ds(16 * t, 16)] for t in range(4)]

        def gstart(g, b):
            for (off, n) in subs:
                pltpu.async_copy(
                    tbl_hbm.at[idx_v.at[pl.ds(g * H + off, n)]],
                    rows_bufs[b].at[pl.ds(off, n)],
                    gsem[b],
                )

        def drain(sem, b):
            pltpu.make_async_copy(
                tbl_hbm.at[pl.ds(0, H)], rows_bufs[b], sem
            ).wait()

        def compute(rows_ref):
            def row_body(r, c):
                v0 = rows_ref[r, pl.ds(0, 16)]
                v1 = rows_ref[r, pl.ds(16, 16)]
                v2 = rows_ref[r, pl.ds(32, 16)]
                v3 = rows_ref[r, pl.ds(48, 16)]
                s = (v0 + v1) + (v2 + v3)
                q = (v0 * v0 + v1 * v1) + (v2 * v2 + v3 * v3)
                for p in perms:
                    s = s + jnp.take_along_axis(s, p, axis=0,
                                                mode="promise_in_bounds")
                    q = q + jnp.take_along_axis(q, p, axis=0,
                                                mode="promise_in_bounds")
                mean = s * (1.0 / D)
                var = q * (1.0 / D) - mean * mean
                xv = var + 1e-5
                ii = plsc.bitcast(xv, jnp.int32)
                ii = 0x5F3759DF - (ii >> 1)
                y = plsc.bitcast(ii, jnp.float32)
                y = y * (1.5 - 0.5 * xv * y * y)
                y = y * (1.5 - 0.5 * xv * y * y)
                y = y * (1.5 - 0.5 * xv * y * y)
                u = mean * y
                for t, vt in enumerate((v0, v1, v2, v3)):
                    rows_ref[r, pl.ds(16 * t, 16)] = \
                        (vt * y - u) * wvecs[t] + bvecs[t]
                return c

            lax.fori_loop(0, H, row_body, 0, unroll=4)

        gstart(0, 0)
        gstart(1, 1)

        def outer(i, carry):
            for b in range(NBUF):
                g = i * NBUF + b
                nb2 = (b + 2) % NBUF

                @pl.when(g >= 2)
                def _():
                    drain(osem[nb2], nb2)

                @pl.when(g + 2 < nb_w)
                def _():
                    gstart(g + 2, nb2)

                drain(gsem[b], b)
                compute(rows_bufs[b])
                pltpu.async_copy(
                    rows_bufs[b],
                    out_hbm.at[b0 + g],
                    osem[b],
                )
            return carry

        lax.fori_loop(0, nb_w // NBUF, outer, 0)
        for g in range(nb_w - 2, nb_w):
            drain(osem[g % NBUF], g % NBUF)

    return sc_kernel(x1, table, ln_weight, ln_bias)
